# baseline (speedup 1.0000x reference)
import functools

import jax
import jax.numpy as jnp
from jax import lax
from jax.experimental import pallas as pl
from jax.experimental.pallas import tpu as pltpu
from jax.experimental.pallas import tpu_sc as plsc

NWAY = 64
KSHOT = 8
QSHOT = 128
K = 32
DIM = 768
S = NWAY * KSHOT
Q = NWAY * QSHOT

RB = 128
CB = 1024
RGRID = S // RB
CGRID = Q // CB
NCHUNK_TK = RB // 8
CPS = NCHUNK_TK // CGRID

_SC_CORES = 2
_SC_SUBCORES = 16
_NW = _SC_CORES * _SC_SUBCORES
_B = S * K
_B_PER_W = _B // _NW
_CHUNK = 64
_NCHUNK = _B_PER_W // _CHUNK


def _simtopk_body(s_ref, q_ref, idx_ref, acc_ref, sim_ref):
    i = pl.program_id(0)
    j = pl.program_id(1)

    sim = lax.dot_general(
        s_ref[...], q_ref[...], (((1,), (1,)), ((), ())),
        preferred_element_type=jnp.float32,
    )
    sim_ref[i & 1, j] = sim

    pb = (i - 1) & 1
    colsf = lax.broadcasted_iota(jnp.int32, (8, Q), 1).astype(jnp.float32)
    part = jnp.float32(0.0)
    for t in range(CPS):
        rc = j * CPS + t
        sim8 = jnp.concatenate(
            [sim_ref[pb, c, pl.ds(rc * 8, 8), :] for c in range(CGRID)],
            axis=1,
        )
        iv_cols = []
        for _ in range(K):
            mv = jnp.max(sim8, axis=1, keepdims=True)
            ivf = jnp.min(
                jnp.where(sim8 == mv, colsf, jnp.float32(Q)),
                axis=1, keepdims=True)
            iv_cols.append(ivf)
            sim8 = jnp.where(colsf == ivf, -jnp.inf, sim8)
        idx_blk = jnp.concatenate(iv_cols, axis=1).astype(jnp.int32)
        idx_ref[pl.ds(rc * 8, 8), :] = idx_blk
        lo = ((i - 1) * (RB // KSHOT) + rc) * QSHOT
        within = (idx_blk >= lo) & (idx_blk < lo + QSHOT)
        part += jnp.sum(within.astype(jnp.float32)) / jnp.float32(S * K)

    @pl.when(j == 0)
    def _init():
        acc_ref[0, 0, 0] = part

    @pl.when(j != 0)
    def _add():
        acc_ref[0, 0, 0] += part


_simtopk = pl.pallas_call(
    _simtopk_body,
    grid=(RGRID + 1, CGRID),
    in_specs=[
        pl.BlockSpec((RB, DIM),
                     lambda i, j: (jnp.minimum(i, RGRID - 1), 0)),
        pl.BlockSpec((CB, DIM),
                     lambda i, j: (jnp.where(i < RGRID, j, 0), 0)),
    ],
    out_specs=[
        pl.BlockSpec((RB, K), lambda i, j: (jnp.maximum(i - 1, 0), 0)),
        pl.BlockSpec((1, 1, 1),
                     lambda i, j: (jnp.maximum(i - 1, 0), 0, 0),
                     memory_space=pltpu.SMEM),
    ],
    out_shape=[
        jax.ShapeDtypeStruct((S, K), jnp.int32),
        jax.ShapeDtypeStruct((RGRID, 1, 1), jnp.float32),
    ],
    scratch_shapes=[pltpu.VMEM((2, CGRID, RB, CB), jnp.float32)],
    compiler_params=pltpu.CompilerParams(
        dimension_semantics=("arbitrary", "arbitrary"),
    ),
)


def _sc_gather_body(table_hbm, idx_hbm, out_hbm, idx_v, rows_v, sem0, sem1):
    wid = lax.axis_index("s") * _SC_CORES + lax.axis_index("c")
    base = wid * _B_PER_W
    pltpu.sync_copy(idx_hbm.at[pl.ds(wid * _NCHUNK, _NCHUNK)], idx_v)
    sems = (sem0, sem1)
    copies = [None, None]
    copies[0] = pltpu.async_copy(
        table_hbm.at[idx_v.at[0]], rows_v.at[0], sems[0])
    for c in range(_NCHUNK):
        if c + 1 < _NCHUNK:
            copies[(c + 1) % 2] = pltpu.async_copy(
                table_hbm.at[idx_v.at[c + 1]], rows_v.at[(c + 1) % 2],
                sems[(c + 1) % 2])
        copies[c % 2].wait()
        pltpu.sync_copy(rows_v.at[c % 2],
                        out_hbm.at[pl.ds(base + c * _CHUNK, _CHUNK)])


@functools.lru_cache(maxsize=1)
def _sc_gather():
    return pl.kernel(
        _sc_gather_body,
        out_type=jax.ShapeDtypeStruct((_B, DIM), jnp.float32),
        mesh=plsc.VectorSubcoreMesh(
            core_axis_name="c", subcore_axis_name="s", num_cores=_SC_CORES),
        scratch_types=[
            pltpu.VMEM((_NCHUNK, _CHUNK), jnp.int32),
            pltpu.VMEM((2, _CHUNK, DIM), jnp.float32),
            pltpu.SemaphoreType.DMA,
            pltpu.SemaphoreType.DMA,
        ],
    )


def kernel(support_embddings, query_embeddings):
    s = support_embddings
    q = query_embeddings
    sn = s / jnp.maximum(
        jnp.linalg.norm(s, ord=2, axis=1, keepdims=True), 1e-12)
    qn = q / jnp.maximum(
        jnp.linalg.norm(q, ord=2, axis=1, keepdims=True), 1e-12)
    nidx, acc_parts = _simtopk(sn, qn)
    gathered = _sc_gather()(q, nidx.reshape(_B // _CHUNK, _CHUNK))
    return gathered.reshape(NWAY, KSHOT * K, DIM), jnp.sum(acc_parts)


# speedup vs baseline: 1.3699x; 1.3699x over previous
import functools

import jax
import jax.numpy as jnp
from jax import lax
from jax.experimental import pallas as pl
from jax.experimental.pallas import tpu as pltpu
from jax.experimental.pallas import tpu_sc as plsc

NWAY = 64
KSHOT = 8
QSHOT = 128
K = 32
DIM = 768
S = NWAY * KSHOT
Q = NWAY * QSHOT

RB = 256
CB = 1024
RGRID = S // RB
CGRID = Q // CB

_SC_CORES = 2
_SC_SUBCORES = 16
_NW = _SC_CORES * _SC_SUBCORES
_B = S * K
_B_PER_W = _B // _NW
_CHUNK = 64
_NCHUNK = _B_PER_W // _CHUNK


def _simtopk_body(s_ref, q_ref, idx_ref, acc_ref, sim_ref):
    i = pl.program_id(0)
    j = pl.program_id(1)

    sim = lax.dot_general(
        s_ref[...], q_ref[...], (((1,), (1,)), ((), ())),
        preferred_element_type=jnp.float32,
    )
    sim_ref[j] = sim

    @pl.when(j == CGRID - 1)
    def _topk():
        total = jnp.float32(0.0)
        colsf = lax.broadcasted_iota(
            jnp.int32, (8, Q), 1).astype(jnp.float32)
        for rc in range(RB // 8):
            sim8 = jnp.concatenate(
                [sim_ref[c, pl.ds(rc * 8, 8), :] for c in range(CGRID)],
                axis=1,
            )
            iv_cols = []
            for k in range(K):
                mv = jnp.max(sim8, axis=1, keepdims=True)
                ivf = jnp.min(
                    jnp.where(sim8 == mv, colsf, jnp.float32(Q)),
                    axis=1, keepdims=True)
                iv_cols.append(ivf)
                if k + 1 < K:
                    sim8 = jnp.where(colsf == ivf, -jnp.inf, sim8)
            idx_blk = jnp.concatenate(iv_cols, axis=1).astype(jnp.int32)
            idx_ref[pl.ds(rc * 8, 8), :] = idx_blk
            lo = (i * (RB // KSHOT) + rc) * QSHOT
            within = (idx_blk >= lo) & (idx_blk < lo + QSHOT)
            total += jnp.sum(within.astype(jnp.float32))

        acc_ref[0, 0, 0] = total / jnp.float32(S * K)


_simtopk = pl.pallas_call(
    _simtopk_body,
    grid=(RGRID, CGRID),
    in_specs=[
        pl.BlockSpec((RB, DIM), lambda i, j: (i, 0)),
        pl.BlockSpec((CB, DIM), lambda i, j: (j, 0)),
    ],
    out_specs=[
        pl.BlockSpec((RB, K), lambda i, j: (i, 0)),
        pl.BlockSpec((1, 1, 1), lambda i, j: (i, 0, 0),
                     memory_space=pltpu.SMEM),
    ],
    out_shape=[
        jax.ShapeDtypeStruct((S, K), jnp.int32),
        jax.ShapeDtypeStruct((RGRID, 1, 1), jnp.float32),
    ],
    scratch_shapes=[pltpu.VMEM((CGRID, RB, CB), jnp.float32)],
    compiler_params=pltpu.CompilerParams(
        dimension_semantics=("arbitrary", "arbitrary"),
    ),
)


def _sc_gather_body(table_hbm, idx_hbm, out_hbm, idx_v, rows_v, sem0, sem1):
    wid = lax.axis_index("s") * _SC_CORES + lax.axis_index("c")
    base = wid * _B_PER_W
    pltpu.sync_copy(idx_hbm.at[pl.ds(wid * _NCHUNK, _NCHUNK)], idx_v)
    sems = (sem0, sem1)
    copies = [None, None]
    copies[0] = pltpu.async_copy(
        table_hbm.at[idx_v.at[0]], rows_v.at[0], sems[0])
    for c in range(_NCHUNK):
        if c + 1 < _NCHUNK:
            copies[(c + 1) % 2] = pltpu.async_copy(
                table_hbm.at[idx_v.at[c + 1]], rows_v.at[(c + 1) % 2],
                sems[(c + 1) % 2])
        copies[c % 2].wait()
        pltpu.sync_copy(rows_v.at[c % 2],
                        out_hbm.at[pl.ds(base + c * _CHUNK, _CHUNK)])


@functools.lru_cache(maxsize=1)
def _sc_gather():
    return pl.kernel(
        _sc_gather_body,
        out_type=jax.ShapeDtypeStruct((_B, DIM), jnp.float32),
        mesh=plsc.VectorSubcoreMesh(
            core_axis_name="c", subcore_axis_name="s", num_cores=_SC_CORES),
        scratch_types=[
            pltpu.VMEM((_NCHUNK, _CHUNK), jnp.int32),
            pltpu.VMEM((2, _CHUNK, DIM), jnp.float32),
            pltpu.SemaphoreType.DMA,
            pltpu.SemaphoreType.DMA,
        ],
    )


def kernel(support_embddings, query_embeddings):
    s = support_embddings
    q = query_embeddings
    sn = s / jnp.maximum(
        jnp.linalg.norm(s, ord=2, axis=1, keepdims=True), 1e-12)
    qn = q / jnp.maximum(
        jnp.linalg.norm(q, ord=2, axis=1, keepdims=True), 1e-12)
    nidx, acc_parts = _simtopk(sn, qn)
    gathered = _sc_gather()(q, nidx.reshape(_B // _CHUNK, _CHUNK))
    return gathered.reshape(NWAY, KSHOT * K, DIM), jnp.sum(acc_parts)


# speedup vs baseline: 1.6442x; 1.2002x over previous
import functools

import jax
import jax.numpy as jnp
from jax import lax
from jax.experimental import pallas as pl
from jax.experimental.pallas import tpu as pltpu
from jax.experimental.pallas import tpu_sc as plsc

NWAY = 64
KSHOT = 8
QSHOT = 128
K = 32
DIM = 768
S = NWAY * KSHOT
Q = NWAY * QSHOT

RB = 512
CB = 1024
RGRID = S // RB
CGRID = Q // CB

_SC_CORES = 2
_SC_SUBCORES = 16
_NW = _SC_CORES * _SC_SUBCORES
_B = S * K
_B_PER_W = _B // _NW
_CHUNK = 64
_NCHUNK = _B_PER_W // _CHUNK


def _simtopk_body(s_ref, q_ref, idx_ref, acc_ref, sim_ref):
    i = pl.program_id(0)
    j = pl.program_id(1)

    sim = lax.dot_general(
        s_ref[...], q_ref[...], (((1,), (1,)), ((), ())),
        preferred_element_type=jnp.float32,
    )
    sim_ref[j] = sim

    @pl.when(j == CGRID - 1)
    def _topk():
        total = jnp.float32(0.0)
        colsf = lax.broadcasted_iota(
            jnp.int32, (8, Q), 1).astype(jnp.float32)
        for rc in range(RB // 8):
            sim8 = jnp.concatenate(
                [sim_ref[c, pl.ds(rc * 8, 8), :] for c in range(CGRID)],
                axis=1,
            )
            iv_cols = []
            for k in range(K):
                mv = jnp.max(sim8, axis=1, keepdims=True)
                ivf = jnp.min(
                    jnp.where(sim8 == mv, colsf, jnp.float32(Q)),
                    axis=1, keepdims=True)
                iv_cols.append(ivf)
                if k + 1 < K:
                    sim8 = jnp.where(colsf == ivf, -jnp.inf, sim8)
            idx_blk = jnp.concatenate(iv_cols, axis=1).astype(jnp.int32)
            idx_ref[pl.ds(rc * 8, 8), :] = idx_blk
            lo = (i * (RB // KSHOT) + rc) * QSHOT
            within = (idx_blk >= lo) & (idx_blk < lo + QSHOT)
            total += jnp.sum(within.astype(jnp.float32))

        acc_ref[0, 0, 0] = total / jnp.float32(S * K)


_simtopk = pl.pallas_call(
    _simtopk_body,
    grid=(RGRID, CGRID),
    in_specs=[
        pl.BlockSpec((RB, DIM), lambda i, j: (i, 0)),
        pl.BlockSpec((CB, DIM), lambda i, j: (j, 0)),
    ],
    out_specs=[
        pl.BlockSpec((RB, K), lambda i, j: (i, 0)),
        pl.BlockSpec((1, 1, 1), lambda i, j: (i, 0, 0),
                     memory_space=pltpu.SMEM),
    ],
    out_shape=[
        jax.ShapeDtypeStruct((S, K), jnp.int32),
        jax.ShapeDtypeStruct((RGRID, 1, 1), jnp.float32),
    ],
    scratch_shapes=[pltpu.VMEM((CGRID, RB, CB), jnp.float32)],
    compiler_params=pltpu.CompilerParams(
        dimension_semantics=("arbitrary", "arbitrary"),
    ),
)


def _sc_gather_body(table_hbm, idx_hbm, out_hbm, idx_v, rows_v, sem0, sem1):
    wid = lax.axis_index("s") * _SC_CORES + lax.axis_index("c")
    base = wid * _B_PER_W
    pltpu.sync_copy(idx_hbm.at[pl.ds(wid * _NCHUNK, _NCHUNK)], idx_v)
    sems = (sem0, sem1)
    copies = [None, None]
    copies[0] = pltpu.async_copy(
        table_hbm.at[idx_v.at[0]], rows_v.at[0], sems[0])
    for c in range(_NCHUNK):
        if c + 1 < _NCHUNK:
            copies[(c + 1) % 2] = pltpu.async_copy(
                table_hbm.at[idx_v.at[c + 1]], rows_v.at[(c + 1) % 2],
                sems[(c + 1) % 2])
        copies[c % 2].wait()
        pltpu.sync_copy(rows_v.at[c % 2],
                        out_hbm.at[pl.ds(base + c * _CHUNK, _CHUNK)])


@functools.lru_cache(maxsize=1)
def _sc_gather():
    return pl.kernel(
        _sc_gather_body,
        out_type=jax.ShapeDtypeStruct((_B, DIM), jnp.float32),
        mesh=plsc.VectorSubcoreMesh(
            core_axis_name="c", subcore_axis_name="s", num_cores=_SC_CORES),
        scratch_types=[
            pltpu.VMEM((_NCHUNK, _CHUNK), jnp.int32),
            pltpu.VMEM((2, _CHUNK, DIM), jnp.float32),
            pltpu.SemaphoreType.DMA,
            pltpu.SemaphoreType.DMA,
        ],
    )


def kernel(support_embddings, query_embeddings):
    s = support_embddings
    q = query_embeddings
    sn = s / jnp.maximum(
        jnp.linalg.norm(s, ord=2, axis=1, keepdims=True), 1e-12)
    qn = q / jnp.maximum(
        jnp.linalg.norm(q, ord=2, axis=1, keepdims=True), 1e-12)
    nidx, acc_parts = _simtopk(sn, qn)
    gathered = _sc_gather()(q, nidx.reshape(_B // _CHUNK, _CHUNK))
    return gathered.reshape(NWAY, KSHOT * K, DIM), jnp.sum(acc_parts)
